# baseline (speedup 1.0000x reference)
import functools

import jax
import jax.numpy as jnp
from jax.experimental import pallas as pl
from jax.experimental.pallas import tpu as pltpu

_MASK_IDX = 5
_CH = 512
_NBUF = 8


def _patch_mm(x_hbm, w_ref, b_ref, acts_ref, o_hbm,
              xbuf, obuf, wc, insem, outsem, *, nch, chunks_per_batch):
    wc[...] = w_ref[...].astype(jnp.bfloat16)

    for s in range(_NBUF):
        pltpu.make_async_copy(
            x_hbm.at[pl.ds(s * _CH, _CH), :], xbuf.at[s], insem.at[s]
        ).start()

    def step(i, carry):
        s = jax.lax.rem(i, _NBUF)
        pltpu.make_async_copy(
            x_hbm.at[pl.ds(i * _CH, _CH), :], xbuf.at[s], insem.at[s]
        ).wait()
        y = jnp.dot(
            xbuf[s].astype(jnp.bfloat16), wc[...],
            preferred_element_type=jnp.float32,
        ) + b_ref[...]

        @pl.when(i >= _NBUF)
        def _():
            pltpu.make_async_copy(
                obuf.at[s], o_hbm.at[pl.ds((i - _NBUF) * _CH, _CH), :],
                outsem.at[s],
            ).wait()

        obuf[s] = y

        @pl.when(jax.lax.rem(i, chunks_per_batch) == 0)
        def _():
            obuf[s, _MASK_IDX, :] = acts_ref[0]

        pltpu.make_async_copy(
            obuf.at[s], o_hbm.at[pl.ds(i * _CH, _CH), :], outsem.at[s]
        ).start()

        @pl.when(i + _NBUF < nch)
        def _():
            pltpu.make_async_copy(
                x_hbm.at[pl.ds((i + _NBUF) * _CH, _CH), :], xbuf.at[s],
                insem.at[s],
            ).start()

        return carry

    jax.lax.fori_loop(0, nch, step, 0)

    for s in range(_NBUF):
        i = nch - _NBUF + s
        sl = jax.lax.rem(i, _NBUF)
        pltpu.make_async_copy(
            obuf.at[sl], o_hbm.at[pl.ds(i * _CH, _CH), :], outsem.at[sl]
        ).wait()


def kernel(x, W, b, acts):
    B, S, D = x.shape
    xf = x.reshape(B * S, D)
    b2 = b.reshape(1, D)
    acts2 = acts.reshape(1, D)
    nch = B * S // _CH
    out = pl.pallas_call(
        functools.partial(_patch_mm, nch=nch, chunks_per_batch=S // _CH),
        in_specs=[
            pl.BlockSpec(memory_space=pl.ANY),
            pl.BlockSpec((D, D), lambda: (0, 0)),
            pl.BlockSpec((1, D), lambda: (0, 0)),
            pl.BlockSpec((1, D), lambda: (0, 0)),
        ],
        out_specs=pl.BlockSpec(memory_space=pl.ANY),
        out_shape=jax.ShapeDtypeStruct((B * S, D), jnp.float32),
        scratch_shapes=[
            pltpu.VMEM((_NBUF, _CH, D), jnp.float32),
            pltpu.VMEM((_NBUF, _CH, D), jnp.float32),
            pltpu.VMEM((D, D), jnp.bfloat16),
            pltpu.SemaphoreType.DMA((_NBUF,)),
            pltpu.SemaphoreType.DMA((_NBUF,)),
        ],
    )(xf, W, b2, acts2)
    return out.reshape(B, S, D)


# speedup vs baseline: 1.0368x; 1.0368x over previous
import functools

import jax
import jax.numpy as jnp
from jax.experimental import pallas as pl
from jax.experimental.pallas import tpu as pltpu

_MASK_IDX = 5
_CH = 1024
_NBUF = 4


def _patch_mm(x_hbm, w_ref, b_ref, acts_ref, o_hbm,
              xbuf, obuf, wc, insem, outsem, *, nch, chunks_per_batch):
    wc[...] = w_ref[...].astype(jnp.bfloat16)

    for s in range(_NBUF):
        pltpu.make_async_copy(
            x_hbm.at[pl.ds(s * _CH, _CH), :], xbuf.at[s], insem.at[s]
        ).start()

    def step(i, carry):
        s = jax.lax.rem(i, _NBUF)
        pltpu.make_async_copy(
            x_hbm.at[pl.ds(i * _CH, _CH), :], xbuf.at[s], insem.at[s]
        ).wait()
        y = jnp.dot(
            xbuf[s].astype(jnp.bfloat16), wc[...],
            preferred_element_type=jnp.float32,
        ) + b_ref[...]

        @pl.when(i >= _NBUF)
        def _():
            pltpu.make_async_copy(
                obuf.at[s], o_hbm.at[pl.ds((i - _NBUF) * _CH, _CH), :],
                outsem.at[s],
            ).wait()

        obuf[s] = y

        @pl.when(jax.lax.rem(i, chunks_per_batch) == 0)
        def _():
            obuf[s, _MASK_IDX, :] = acts_ref[0]

        pltpu.make_async_copy(
            obuf.at[s], o_hbm.at[pl.ds(i * _CH, _CH), :], outsem.at[s]
        ).start()

        @pl.when(i + _NBUF < nch)
        def _():
            pltpu.make_async_copy(
                x_hbm.at[pl.ds((i + _NBUF) * _CH, _CH), :], xbuf.at[s],
                insem.at[s],
            ).start()

        return carry

    jax.lax.fori_loop(0, nch, step, 0)

    for s in range(_NBUF):
        i = nch - _NBUF + s
        sl = jax.lax.rem(i, _NBUF)
        pltpu.make_async_copy(
            obuf.at[sl], o_hbm.at[pl.ds(i * _CH, _CH), :], outsem.at[sl]
        ).wait()


def kernel(x, W, b, acts):
    B, S, D = x.shape
    xf = x.reshape(B * S, D)
    b2 = b.reshape(1, D)
    acts2 = acts.reshape(1, D)
    nch = B * S // _CH
    out = pl.pallas_call(
        functools.partial(_patch_mm, nch=nch, chunks_per_batch=S // _CH),
        in_specs=[
            pl.BlockSpec(memory_space=pl.ANY),
            pl.BlockSpec((D, D), lambda: (0, 0)),
            pl.BlockSpec((1, D), lambda: (0, 0)),
            pl.BlockSpec((1, D), lambda: (0, 0)),
        ],
        out_specs=pl.BlockSpec(memory_space=pl.ANY),
        out_shape=jax.ShapeDtypeStruct((B * S, D), jnp.float32),
        scratch_shapes=[
            pltpu.VMEM((_NBUF, _CH, D), jnp.float32),
            pltpu.VMEM((_NBUF, _CH, D), jnp.float32),
            pltpu.VMEM((D, D), jnp.bfloat16),
            pltpu.SemaphoreType.DMA((_NBUF,)),
            pltpu.SemaphoreType.DMA((_NBUF,)),
        ],
    )(xf, W, b2, acts2)
    return out.reshape(B, S, D)
